# baseline (speedup 1.0000x reference)
import jax
import jax.numpy as jnp
from jax import lax
from jax.experimental import pallas as pl
from jax.experimental.pallas import tpu as pltpu
from jax.experimental.pallas import tpu_sc as plsc

B = 4096
L = 200
WDIM = 64
PDIM = 16
ODIM = WDIM + 2 * PDIM
NC = 2
NS = 16
NW = NC * NS
ROWS = B // NW
RS = 2
T = RS * L
STEPS = ROWS // RS


def _emb_body(word_hbm, pos1_hbm, pos2_hbm, wtab_hbm, p1tab_hbm, p2tab_hbm,
              out_hbm, widx_v, p1idx_v, p2idx_v, wrows_v, p1rows_v, p2rows_v,
              sem_i, sem_g, sem_o):
    c = lax.axis_index("c")
    s = lax.axis_index("s")
    wid = s * NC + c
    row0 = wid * ROWS

    def idx_copies(i, p):
        bb = row0 + i * RS
        return [
            pltpu.make_async_copy(word_hbm.at[pl.ds(bb, RS)],
                                  widx_v.at[p], sem_i.at[p]),
            pltpu.make_async_copy(pos1_hbm.at[pl.ds(bb, RS)],
                                  p1idx_v.at[p], sem_i.at[p]),
            pltpu.make_async_copy(pos2_hbm.at[pl.ds(bb, RS)],
                                  p2idx_v.at[p], sem_i.at[p]),
        ]

    def gather_copies(p):
        cps = []
        for j in range(RS):
            cps.append(pltpu.make_async_copy(
                wtab_hbm.at[widx_v.at[p, j]], wrows_v.at[p, j], sem_g.at[p]))
            cps.append(pltpu.make_async_copy(
                p1tab_hbm.at[p1idx_v.at[p, j]], p1rows_v.at[p, j],
                sem_g.at[p]))
            cps.append(pltpu.make_async_copy(
                p2tab_hbm.at[p2idx_v.at[p, j]], p2rows_v.at[p, j],
                sem_g.at[p]))
        return cps

    def out_copies(i, p):
        bb = row0 + i * RS
        return [
            pltpu.make_async_copy(
                wrows_v.at[p],
                out_hbm.at[pl.ds(bb, RS), pl.ds(0, L), pl.ds(0, WDIM)],
                sem_o.at[p]),
            pltpu.make_async_copy(
                p1rows_v.at[p],
                out_hbm.at[pl.ds(bb, RS), pl.ds(0, L), pl.ds(WDIM, PDIM)],
                sem_o.at[p]),
            pltpu.make_async_copy(
                p2rows_v.at[p],
                out_hbm.at[pl.ds(bb, RS), pl.ds(0, L),
                           pl.ds(WDIM + PDIM, PDIM)],
                sem_o.at[p]),
        ]

    def fire(copies):
        for cp in copies:
            cp.start()

    def drain(copies):
        for cp in copies:
            cp.wait()

    fire(idx_copies(0, 0))
    fire(idx_copies(1, 1))
    drain(idx_copies(0, 0))
    fire(gather_copies(0))

    def step(i, carry):
        p = i & 1
        q = 1 - p
        drain(gather_copies(q))
        fire(out_copies(i - 1, q))

        @pl.when(i + 1 < STEPS)
        def _():
            fire(idx_copies(i + 1, q))

        drain(idx_copies(i, p))

        @pl.when(i >= 2)
        def _():
            drain(out_copies(i - 2, p))

        fire(gather_copies(p))
        return carry

    lax.fori_loop(1, STEPS, step, 0)

    qe = (STEPS - 1) & 1
    drain(gather_copies(qe))
    fire(out_copies(STEPS - 1, qe))
    drain(out_copies(STEPS - 2, 1 - qe))
    drain(out_copies(STEPS - 1, qe))


@jax.jit
def _run(word, pos1, pos2, word_table, p1_tab, p2_tab):
    mesh = plsc.VectorSubcoreMesh(core_axis_name="c", subcore_axis_name="s")
    f = pl.kernel(
        _emb_body,
        mesh=mesh,
        compiler_params=pltpu.CompilerParams(use_tc_tiling_on_sc=False),
        out_type=jax.ShapeDtypeStruct((B, L, ODIM), jnp.float32),
        scratch_types=[
            pltpu.VMEM((2, RS, L), jnp.int32),
            pltpu.VMEM((2, RS, L), jnp.int32),
            pltpu.VMEM((2, RS, L), jnp.int32),
            pltpu.VMEM((2, RS, L, WDIM), jnp.float32),
            pltpu.VMEM((2, RS, L, PDIM), jnp.float32),
            pltpu.VMEM((2, RS, L, PDIM), jnp.float32),
            pltpu.SemaphoreType.DMA((2,)),
            pltpu.SemaphoreType.DMA((2,)),
            pltpu.SemaphoreType.DMA((2,)),
        ],
    )
    return f(word, pos1, pos2, word_table, p1_tab, p2_tab)


def kernel(word, pos1, pos2, word_table, pos1_table, pos2_table):
    word = word.astype(jnp.int32)
    pos1 = pos1.astype(jnp.int32)
    pos2 = pos2.astype(jnp.int32)
    p1_tab = pos1_table.at[0].set(0.0)
    p2_tab = pos2_table.at[0].set(0.0)
    return _run(word, pos1, pos2, word_table, p1_tab, p2_tab)
